# Initial kernel scaffold; baseline (speedup 1.0000x reference)
import jax
import jax.numpy as jnp
from jax.experimental import pallas as pl


def kernel(x, edge_index, edge_weight, prev_hidden_state, Wz, bz, Lz, lbz, Wr, br, Lr, lbr, Wh, bh, Lh, lbh, Wlin, blin):
    raise NotImplementedError("write your pallas kernel here")


# speedup vs baseline: 11.7315x; 11.7315x over previous
import functools

import jax
import jax.numpy as jnp
from jax import lax
from jax.experimental import pallas as pl
from jax.experimental.pallas import tpu as pltpu
from jax.experimental.pallas import tpu_sc as plsc

_L = 16
_NT = 16
_NC = 2
_CHUNK = 128
_ROWH = 128


def _rsqrt_vec(d):
    xi = lax.bitcast_convert_type(d, jnp.int32)
    yi = jnp.int32(0x5F3759DF) - (xi >> 1)
    y = lax.bitcast_convert_type(yi, jnp.float32)
    for _ in range(3):
        y = y * (1.5 - 0.5 * d * y * y)
    return y


def _sc_body(nch, rpt, src_h, dst_h, ew_h, xv_h, out_h,
             accum, deg_s, dinv_t, rowbuf, src_b, src2_b, dst_b, ew_b,
             normbuf, tmp, sem):
    c = lax.axis_index("c")
    s = lax.axis_index("s")
    base = s * rpt
    zvec = jnp.zeros((_L,), jnp.float32)

    def ztmp(i, _):
        tmp[pl.ds(i * _L, _L)] = zvec
        return 0
    lax.fori_loop(0, rpt // _L, ztmp, 0)

    def zrow(i, _):
        for u in range(_ROWH // _L):
            rowbuf[i, pl.ds(u * _L, _L)] = zvec
        return 0
    lax.fori_loop(0, _CHUNK, zrow, 0)

    pltpu.sync_copy(tmp, deg_s.at[pl.ds(base, rpt)])
    for k in range(rpt // _CHUNK):
        pltpu.sync_copy(rowbuf, accum.at[pl.ds(base + k * _CHUNK, _CHUNK)])

    plsc.subcore_barrier()

    def fdeg(j, _):
        pltpu.sync_copy(ew_h.at[s, j], ew_b)
        pltpu.sync_copy(dst_h.at[s, j], dst_b)
        pltpu.sync_copy(ew_b, deg_s.at[dst_b], add=True)
        return 0
    lax.fori_loop(0, nch, fdeg, 0)

    plsc.subcore_barrier()

    pltpu.sync_copy(deg_s.at[pl.ds(base, rpt)], tmp)

    def fnewton(i, _):
        d = tmp[pl.ds(i * _L, _L)]
        tmp[pl.ds(i * _L, _L)] = _rsqrt_vec(d)
        return 0
    lax.fori_loop(0, rpt // _L, fnewton, 0)
    pltpu.sync_copy(tmp, deg_s.at[pl.ds(base, rpt)])
    plsc.subcore_barrier()
    pltpu.sync_copy(deg_s, dinv_t)

    cc = jnp.full((_L,), 0, jnp.int32) + c

    def fmain(j, _):
        pltpu.sync_copy(src_h.at[s, j], src_b)
        pltpu.sync_copy(dst_h.at[s, j], dst_b)
        pltpu.sync_copy(ew_h.at[s, j], ew_b)
        for u in range(_CHUNK // _L):
            sl = pl.ds(u * _L, _L)
            src2_b[sl] = src_b[sl] * 2 + cc
        pltpu.async_copy(xv_h.at[src2_b], rowbuf, sem).wait()
        for u in range(_CHUNK // _L):
            sl = pl.ds(u * _L, _L)
            ns = plsc.load_gather(dinv_t, [src_b[sl]])
            nd = plsc.load_gather(dinv_t, [dst_b[sl]])
            normbuf[sl] = ns * ew_b[sl] * nd

        def fscale(e, _2):
            nb = plsc.load_gather(normbuf, [jnp.full((_L,), 0, jnp.int32) + e])
            for u in range(_ROWH // _L):
                sl = (e, pl.ds(u * _L, _L))
                rowbuf[sl] = rowbuf[sl] * nb
            return 0
        lax.fori_loop(0, _CHUNK, fscale, 0)
        pltpu.sync_copy(rowbuf, accum.at[dst_b], add=True)
        return 0
    lax.fori_loop(0, nch, fmain, 0)

    plsc.subcore_barrier()
    pltpu.sync_copy(accum.at[pl.ds(base, rpt)], out_h.at[c, pl.ds(base, rpt)])


def _sc_call(src3, dst3, ew3, xview, nch, npad):
    rpt = npad // _NT
    mesh = plsc.VectorSubcoreMesh(core_axis_name="c", subcore_axis_name="s")
    f = pl.kernel(
        functools.partial(_sc_body, nch, rpt),
        mesh=mesh,
        compiler_params=pltpu.CompilerParams(needs_layout_passes=False),
        out_type=jax.ShapeDtypeStruct((_NC, npad, _ROWH), jnp.float32),
        scratch_types=[
            pltpu.VMEM_SHARED((npad, _ROWH), jnp.float32),
            pltpu.VMEM_SHARED((npad,), jnp.float32),
            pltpu.VMEM((npad,), jnp.float32),
            pltpu.VMEM((_CHUNK, _ROWH), jnp.float32),
            pltpu.VMEM((_CHUNK,), jnp.int32),
            pltpu.VMEM((_CHUNK,), jnp.int32),
            pltpu.VMEM((_CHUNK,), jnp.int32),
            pltpu.VMEM((_CHUNK,), jnp.float32),
            pltpu.VMEM((_CHUNK,), jnp.float32),
            pltpu.VMEM((npad // _NT,), jnp.float32),
            pltpu.SemaphoreType.DMA,
        ],
    )
    return f(src3, dst3, ew3, xview)


def _tc_body(xa_ref, h_ref, wz, bz, lz0, lz1, lbz, wr, br, lr0, lr1, lbr,
             wh, bh, lh0, lh1, lbh, wlin, blin, y_ref, hn_ref):
    dot = lambda a, b: jnp.dot(a, b, preferred_element_type=jnp.float32)
    xaf = jnp.concatenate([xa_ref[0], xa_ref[1]], axis=1)
    h = h_ref[...]
    z = jax.nn.sigmoid(dot(dot(xaf, wz[...]) + bz[...], lz0[...])
                       + dot(h, lz1[...]) + lbz[...])
    r = jax.nn.sigmoid(dot(dot(xaf, wr[...]) + br[...], lr0[...])
                       + dot(h, lr1[...]) + lbr[...])
    ht = jnp.tanh(dot(dot(xaf, wh[...]) + bh[...], lh0[...])
                  + dot(h * r, lh1[...]) + lbh[...])
    hn = z * h + (1.0 - z) * ht
    y_ref[...] = dot(jax.nn.relu(hn), wlin[...]) + blin[...]
    hn_ref[...] = hn


def _tc_call(xa2, h, wz, bz, lz0, lz1, lbz, wr, br, lr0, lr1, lbr,
             wh, bh, lh0, lh1, lbh, wlin, blin):
    n, cdim = h.shape
    rb = 1000
    grid = (n // rb,)
    wspec = pl.BlockSpec((cdim, cdim), lambda i: (0, 0))
    bspec = pl.BlockSpec((1, cdim), lambda i: (0, 0))
    rspec = pl.BlockSpec((rb, cdim), lambda i: (i, 0))
    return pl.pallas_call(
        _tc_body,
        grid=grid,
        in_specs=[
            pl.BlockSpec((_NC, rb, _ROWH), lambda i: (0, i, 0)),
            rspec,
            wspec, bspec, wspec, wspec, bspec,
            wspec, bspec, wspec, wspec, bspec,
            wspec, bspec, wspec, wspec, bspec,
            wspec, bspec,
        ],
        out_specs=[rspec, rspec],
        out_shape=[
            jax.ShapeDtypeStruct((n, cdim), jnp.float32),
            jax.ShapeDtypeStruct((n, cdim), jnp.float32),
        ],
    )(xa2, h, wz, bz, lz0, lz1, lbz, wr, br, lr0, lr1, lbr,
      wh, bh, lh0, lh1, lbh, wlin, blin)


def kernel(x, edge_index, edge_weight, prev_hidden_state,
           Wz, bz, Lz, lbz, Wr, br, Lr, lbr, Wh, bh, Lh, lbh, Wlin, blin):
    n, cdim = x.shape
    src = edge_index[0]
    dst = edge_index[1]
    loop = jnp.arange(n, dtype=src.dtype)

    tot = edge_weight.shape[0] + n
    per_tile = -(-tot // _NT)
    nch = -(-per_tile // _CHUNK)
    pad = _NT * nch * _CHUNK - tot
    npad = -(-n // (_NT * _CHUNK)) * (_NT * _CHUNK)

    srcp = jnp.concatenate([src, loop, jnp.zeros((pad,), src.dtype)]).reshape(_NT, nch, _CHUNK)
    dstp = jnp.concatenate([dst, loop, jnp.zeros((pad,), dst.dtype)]).reshape(_NT, nch, _CHUNK)
    ewp = jnp.concatenate([edge_weight, jnp.ones((n,), edge_weight.dtype),
                           jnp.zeros((pad,), edge_weight.dtype)]).reshape(_NT, nch, _CHUNK)
    xview = x.reshape(2 * n, _ROWH)

    xa2 = _sc_call(srcp, dstp, ewp, xview, nch, npad)
    xa2 = xa2[:, :n, :]

    half = cdim
    y, hn = _tc_call(xa2, prev_hidden_state,
                     Wz, bz.reshape(1, -1), Lz[:half], Lz[half:], lbz.reshape(1, -1),
                     Wr, br.reshape(1, -1), Lr[:half], Lr[half:], lbr.reshape(1, -1),
                     Wh, bh.reshape(1, -1), Lh[:half], Lh[half:], lbh.reshape(1, -1),
                     Wlin, blin.reshape(1, -1))
    return (y, hn)


# speedup vs baseline: 20.3397x; 1.7338x over previous
import functools

import jax
import jax.numpy as jnp
from jax import lax
from jax.experimental import pallas as pl
from jax.experimental.pallas import tpu as pltpu
from jax.experimental.pallas import tpu_sc as plsc

_L = 16
_NT = 16
_NC = 2
_CHUNK = 128
_ROWH = 128


def _rsqrt_vec(d):
    xi = lax.bitcast_convert_type(d, jnp.int32)
    yi = jnp.int32(0x5F3759DF) - (xi >> 1)
    y = lax.bitcast_convert_type(yi, jnp.float32)
    for _ in range(3):
        y = y * (1.5 - 0.5 * d * y * y)
    return y


def _sc_body(nch, rpt, src_h, dst_h, ew_h, xv_h, out_h,
             accum, deg_s, dinv_t, rowb0, rowb1,
             a0s, a0s2, a0d, a0w, a1s, a1s2, a1d, a1w,
             d0d, d0w, d1d, d1w, normbuf, tmp,
             g0, g1, s0, s1, e0, e1, d0, d1):
    c = lax.axis_index("c")
    s = lax.axis_index("s")
    base = s * rpt
    zvec = jnp.zeros((_L,), jnp.float32)
    cc = jnp.full((_L,), 0, jnp.int32) + c
    KD = d0d.shape[0]
    NG = nch // KD

    def dload(db, wb, gi, sem):
        for jj in range(KD):
            pltpu.async_copy(dst_h.at[s, gi * KD + jj], db.at[jj], sem)
            pltpu.async_copy(ew_h.at[s, gi * KD + jj], wb.at[jj], sem)

    def dwait(db, wb, sem):
        for jj in range(KD):
            pltpu.make_async_copy(dst_h.at[s, 0], db.at[jj], sem).wait()
            pltpu.make_async_copy(ew_h.at[s, 0], wb.at[jj], sem).wait()

    dload(d0d, d0w, 0, e0)
    dload(d1d, d1w, 1, e1)

    def ztmp(i, _):
        tmp[pl.ds(i * _L, _L)] = zvec
        return 0
    lax.fori_loop(0, rpt // _L, ztmp, 0)

    def zrow(i, _):
        for u in range(_ROWH // _L):
            rowb0[i, pl.ds(u * _L, _L)] = zvec
        return 0
    lax.fori_loop(0, _CHUNK, zrow, 0)

    pltpu.sync_copy(tmp, deg_s.at[pl.ds(base, rpt)])
    for k in range(rpt // _CHUNK):
        pltpu.sync_copy(rowb0, accum.at[pl.ds(base + k * _CHUNK, _CHUNK)])

    plsc.subcore_barrier()

    def fdeg(gg, _):
        g0i = 2 * gg
        dwait(d0d, d0w, e0)
        hs = [pltpu.async_copy(d0w.at[jj], deg_s.at[d0d.at[jj]], s0, add=True)
              for jj in range(KD)]
        for h in hs:
            h.wait()

        @pl.when(gg < NG // 2 - 1)
        def _():
            dload(d0d, d0w, g0i + 2, e0)

        dwait(d1d, d1w, e1)
        hs1 = [pltpu.async_copy(d1w.at[jj], deg_s.at[d1d.at[jj]], s1, add=True)
               for jj in range(KD)]
        for h in hs1:
            h.wait()

        @pl.when(gg < NG // 2 - 1)
        def _():
            dload(d1d, d1w, g0i + 3, e1)
        return 0
    lax.fori_loop(0, NG // 2, fdeg, 0)

    plsc.subcore_barrier()

    pltpu.sync_copy(deg_s.at[pl.ds(base, rpt)], tmp)

    def fnewton(i, _):
        d = tmp[pl.ds(i * _L, _L)]
        tmp[pl.ds(i * _L, _L)] = _rsqrt_vec(d)
        return 0
    lax.fori_loop(0, rpt // _L, fnewton, 0)
    pltpu.sync_copy(tmp, deg_s.at[pl.ds(base, rpt)])
    plsc.subcore_barrier()
    pltpu.sync_copy(deg_s, dinv_t)

    def norm_scale(sb, db, wb, rowb):
        for u in range(_CHUNK // _L):
            sl = pl.ds(u * _L, _L)
            ns = plsc.load_gather(dinv_t, [sb[sl]])
            nd = plsc.load_gather(dinv_t, [db[sl]])
            normbuf[sl] = ns * wb[sl] * nd

        @plsc.parallel_loop(0, _CHUNK, unroll=2)
        def _(e):
            nb = plsc.load_gather(normbuf, [jnp.full((_L,), 0, jnp.int32) + e])
            for u in range(_ROWH // _L):
                sl = (e, pl.ds(u * _L, _L))
                rowb[sl] = rowb[sl] * nb

    def src2(sb, s2b):
        for u in range(_CHUNK // _L):
            sl = pl.ds(u * _L, _L)
            s2b[sl] = sb[sl] * 2 + cc

    pltpu.sync_copy(src_h.at[s, 0], a0s)
    pltpu.sync_copy(ew_h.at[s, 0], a0w)
    pltpu.sync_copy(dst_h.at[s, 0], a0d)
    pltpu.sync_copy(src_h.at[s, 1], a1s)
    pltpu.sync_copy(ew_h.at[s, 1], a1w)
    src2(a0s, a0s2)
    pltpu.async_copy(xv_h.at[a0s2], rowb0, g0)

    def fmain(t, _):
        j0 = 2 * t
        pltpu.make_async_copy(xv_h.at[pl.ds(0, _CHUNK)], rowb0, g0).wait()
        norm_scale(a0s, a0d, a0w, rowb0)

        @pl.when(t < nch // 2 - 1)
        def _():
            pltpu.async_copy(src_h.at[s, j0 + 2], a0s, e0)
            pltpu.async_copy(ew_h.at[s, j0 + 2], a0w, e0)

        @pl.when(t > 0)
        def _():
            pltpu.make_async_copy(rowb1, accum.at[a1d], s1).wait()
        pltpu.async_copy(dst_h.at[s, j0 + 1], a1d, d1)

        @pl.when(t > 0)
        def _():
            pltpu.make_async_copy(dst_h.at[s, 0], a0d, d0).wait()
        sc0 = pltpu.async_copy(rowb0, accum.at[a0d], s0, add=True)

        @pl.when(t > 0)
        def _():
            pltpu.make_async_copy(src_h.at[s, 0], a1s, e1).wait()
            pltpu.make_async_copy(ew_h.at[s, 0], a1w, e1).wait()
        src2(a1s, a1s2)
        pltpu.async_copy(xv_h.at[a1s2], rowb1, g1)

        pltpu.make_async_copy(xv_h.at[pl.ds(0, _CHUNK)], rowb1, g1).wait()
        norm_scale(a1s, a1d, a1w, rowb1)

        @pl.when(t < nch // 2 - 1)
        def _():
            pltpu.async_copy(src_h.at[s, j0 + 3], a1s, e1)
            pltpu.async_copy(ew_h.at[s, j0 + 3], a1w, e1)

        pltpu.make_async_copy(dst_h.at[s, 0], a1d, d1).wait()
        sc0.wait()
        pltpu.async_copy(rowb1, accum.at[a1d], s1, add=True)

        @pl.when(t < nch // 2 - 1)
        def _():
            pltpu.async_copy(dst_h.at[s, j0 + 2], a0d, d0)
            pltpu.make_async_copy(src_h.at[s, 0], a0s, e0).wait()
            pltpu.make_async_copy(ew_h.at[s, 0], a0w, e0).wait()
            src2(a0s, a0s2)
            pltpu.async_copy(xv_h.at[a0s2], rowb0, g0)
        return 0
    lax.fori_loop(0, nch // 2, fmain, 0)

    pltpu.make_async_copy(rowb1, accum.at[a1d], s1).wait()
    plsc.subcore_barrier()
    pltpu.sync_copy(accum.at[pl.ds(base, rpt)], out_h.at[c, pl.ds(base, rpt)])


def _sc_call(src3, dst3, ew3, xview, nch, npad):
    rpt = npad // _NT
    kd = 3
    mesh = plsc.VectorSubcoreMesh(core_axis_name="c", subcore_axis_name="s")
    f = pl.kernel(
        functools.partial(_sc_body, nch, rpt),
        mesh=mesh,
        compiler_params=pltpu.CompilerParams(needs_layout_passes=False),
        out_type=jax.ShapeDtypeStruct((_NC, npad, _ROWH), jnp.float32),
        scratch_types=[
            pltpu.VMEM_SHARED((npad, _ROWH), jnp.float32),
            pltpu.VMEM_SHARED((npad,), jnp.float32),
            pltpu.VMEM((npad,), jnp.float32),
            pltpu.VMEM((_CHUNK, _ROWH), jnp.float32),
            pltpu.VMEM((_CHUNK, _ROWH), jnp.float32),
            pltpu.VMEM((_CHUNK,), jnp.int32),
            pltpu.VMEM((_CHUNK,), jnp.int32),
            pltpu.VMEM((_CHUNK,), jnp.int32),
            pltpu.VMEM((_CHUNK,), jnp.float32),
            pltpu.VMEM((_CHUNK,), jnp.int32),
            pltpu.VMEM((_CHUNK,), jnp.int32),
            pltpu.VMEM((_CHUNK,), jnp.int32),
            pltpu.VMEM((_CHUNK,), jnp.float32),
            pltpu.VMEM((kd, _CHUNK), jnp.int32),
            pltpu.VMEM((kd, _CHUNK), jnp.float32),
            pltpu.VMEM((kd, _CHUNK), jnp.int32),
            pltpu.VMEM((kd, _CHUNK), jnp.float32),
            pltpu.VMEM((_CHUNK,), jnp.float32),
            pltpu.VMEM((npad // _NT,), jnp.float32),
            pltpu.SemaphoreType.DMA,
            pltpu.SemaphoreType.DMA,
            pltpu.SemaphoreType.DMA,
            pltpu.SemaphoreType.DMA,
            pltpu.SemaphoreType.DMA,
            pltpu.SemaphoreType.DMA,
            pltpu.SemaphoreType.DMA,
            pltpu.SemaphoreType.DMA,
        ],
    )
    return f(src3, dst3, ew3, xview)


def _tc_body(xa_ref, h_ref, wz, bz, lz0, lz1, lbz, wr, br, lr0, lr1, lbr,
             wh, bh, lh0, lh1, lbh, wlin, blin, y_ref, hn_ref):
    dot = lambda a, b: jnp.dot(a, b, preferred_element_type=jnp.float32)
    xaf = jnp.concatenate([xa_ref[0], xa_ref[1]], axis=1)
    h = h_ref[...]
    z = jax.nn.sigmoid(dot(dot(xaf, wz[...]) + bz[...], lz0[...])
                       + dot(h, lz1[...]) + lbz[...])
    r = jax.nn.sigmoid(dot(dot(xaf, wr[...]) + br[...], lr0[...])
                       + dot(h, lr1[...]) + lbr[...])
    ht = jnp.tanh(dot(dot(xaf, wh[...]) + bh[...], lh0[...])
                  + dot(h * r, lh1[...]) + lbh[...])
    hn = z * h + (1.0 - z) * ht
    y_ref[...] = dot(jax.nn.relu(hn), wlin[...]) + blin[...]
    hn_ref[...] = hn


def _tc_call(xa2, h, wz, bz, lz0, lz1, lbz, wr, br, lr0, lr1, lbr,
             wh, bh, lh0, lh1, lbh, wlin, blin):
    n, cdim = h.shape
    rb = 1000
    grid = (n // rb,)
    wspec = pl.BlockSpec((cdim, cdim), lambda i: (0, 0))
    bspec = pl.BlockSpec((1, cdim), lambda i: (0, 0))
    rspec = pl.BlockSpec((rb, cdim), lambda i: (i, 0))
    return pl.pallas_call(
        _tc_body,
        grid=grid,
        in_specs=[
            pl.BlockSpec((_NC, rb, _ROWH), lambda i: (0, i, 0)),
            rspec,
            wspec, bspec, wspec, wspec, bspec,
            wspec, bspec, wspec, wspec, bspec,
            wspec, bspec, wspec, wspec, bspec,
            wspec, bspec,
        ],
        out_specs=[rspec, rspec],
        out_shape=[
            jax.ShapeDtypeStruct((n, cdim), jnp.float32),
            jax.ShapeDtypeStruct((n, cdim), jnp.float32),
        ],
    )(xa2, h, wz, bz, lz0, lz1, lbz, wr, br, lr0, lr1, lbr,
      wh, bh, lh0, lh1, lbh, wlin, blin)


def kernel(x, edge_index, edge_weight, prev_hidden_state,
           Wz, bz, Lz, lbz, Wr, br, Lr, lbr, Wh, bh, Lh, lbh, Wlin, blin):
    n, cdim = x.shape
    src = edge_index[0]
    dst = edge_index[1]
    loop = jnp.arange(n, dtype=src.dtype)

    tot = edge_weight.shape[0] + n
    per_tile = -(-tot // _NT)
    nch = -(-per_tile // _CHUNK)
    pad = _NT * nch * _CHUNK - tot
    npad = -(-n // (_NT * _CHUNK)) * (_NT * _CHUNK)

    srcp = jnp.concatenate([src, loop, jnp.zeros((pad,), src.dtype)]).reshape(_NT, nch, _CHUNK)
    dstp = jnp.concatenate([dst, loop, jnp.zeros((pad,), dst.dtype)]).reshape(_NT, nch, _CHUNK)
    ewp = jnp.concatenate([edge_weight, jnp.ones((n,), edge_weight.dtype),
                           jnp.zeros((pad,), edge_weight.dtype)]).reshape(_NT, nch, _CHUNK)
    xview = x.reshape(2 * n, _ROWH)

    xa2 = _sc_call(srcp, dstp, ewp, xview, nch, npad)
    xa2 = xa2[:, :n, :]

    half = cdim
    y, hn = _tc_call(xa2, prev_hidden_state,
                     Wz, bz.reshape(1, -1), Lz[:half], Lz[half:], lbz.reshape(1, -1),
                     Wr, br.reshape(1, -1), Lr[:half], Lr[half:], lbr.reshape(1, -1),
                     Wh, bh.reshape(1, -1), Lh[:half], Lh[half:], lbh.reshape(1, -1),
                     Wlin, blin.reshape(1, -1))
    return (y, hn)
